# Initial kernel scaffold; baseline (speedup 1.0000x reference)
import jax
import jax.numpy as jnp
from jax.experimental import pallas as pl


def kernel(pooled, node_embeddings, ev_indexes, segment_ids, action_mapper, W1, b1, W2, b2, W3, b3):
    B = pooled.shape[0]
    ACT_DIM = 4096
    T = ev_indexes.shape[0]
    context = jax.nn.relu(pooled[:, 0, :] @ W1 + b1)
    ev_emb = jnp.take(node_embeddings, ev_indexes, axis=0)
    rep_ctx = jnp.take(context, segment_ids, axis=0)
    x = jnp.concatenate([ev_emb, rep_ctx], axis=-1)
    h = jax.nn.relu(x @ W2 + b2)
    vals = (h @ W3 + b3)[:, 0]
    flat = segment_ids * ACT_DIM + action_mapper
    winner = jnp.full((B * ACT_DIM,), -1, dtype=jnp.int32)
    winner = winner.at[flat].max(jnp.arange(T, dtype=jnp.int32))
    out = jnp.where(winner >= 0, jnp.take(vals, jnp.clip(winner, 0, T - 1)), 0.0)
    return jnp.tanh(out.reshape(B, ACT_DIM))


# speedup vs baseline: 4.2712x; 4.2712x over previous
import functools

import jax
import jax.numpy as jnp
from jax import lax
from jax.experimental import pallas as pl
from jax.experimental.pallas import tpu as pltpu
from jax.experimental.pallas import tpu_sc as plsc

B = 16
H = 128
ACT_DIM = 4096
T = 32768
N_NODES = 65536

NC = 2
NS = 16
NW = NC * NS
LANES = 16


ROWS_PER_W = T // NW
GATHER_BATCH = 128
N_BATCH = ROWS_PER_W // GATHER_BATCH

@functools.cache
def _make_sc_gather():
    mesh = plsc.VectorSubcoreMesh(
        core_axis_name="c", subcore_axis_name="s", num_cores=NC, num_subcores=NS)
    return functools.partial(
        pl.kernel,
        out_type=jax.ShapeDtypeStruct((T, H), jnp.float32),
        mesh=mesh,
        compiler_params=pltpu.CompilerParams(needs_layout_passes=False),
        scratch_types=[
            pltpu.VMEM((N_BATCH, GATHER_BATCH), jnp.int32),
            pltpu.VMEM((GATHER_BATCH, H), jnp.float32),
            pltpu.VMEM((GATHER_BATCH, H), jnp.float32),
            pltpu.SemaphoreType.DMA,
            pltpu.SemaphoreType.DMA,
        ],
    )(_sc_gather_body)


def _sc_gather_body(table_hbm, idx_hbm, out_hbm, idx_v, rows_a, rows_b, sem_a, sem_b):
    wid = lax.axis_index("s") * NC + lax.axis_index("c")
    pltpu.sync_copy(idx_hbm.at[pl.ds(wid * N_BATCH, N_BATCH)], idx_v)
    bufs = (rows_a, rows_b)
    sems = (sem_a, sem_b)
    cps = [None, None]
    cps[0] = pltpu.async_copy(table_hbm.at[idx_v.at[0]], bufs[0], sems[0])
    for j in range(N_BATCH):
        if j + 1 < N_BATCH:
            cps[(j + 1) % 2] = pltpu.async_copy(
                table_hbm.at[idx_v.at[j + 1]], bufs[(j + 1) % 2], sems[(j + 1) % 2])
        cps[j % 2].wait()
        pltpu.sync_copy(
            bufs[j % 2],
            out_hbm.at[pl.ds(wid * ROWS_PER_W + j * GATHER_BATCH, GATHER_BATCH)])


TB = 2048


def _tc_mlp_body(ev_ref, seg_ref, pooled_ref, w1_ref, b1_ref, w2a_ref, w2b_ref,
                 b2_ref, w3_ref, b3_ref, out_ref):
    ctx = jax.nn.relu(
        jnp.dot(pooled_ref[...], w1_ref[...], preferred_element_type=jnp.float32)
        + b1_ref[...])
    ctx2 = jnp.dot(ctx, w2b_ref[...], preferred_element_type=jnp.float32) + b2_ref[...]
    seg = seg_ref[...]
    onehot = (lax.broadcasted_iota(jnp.int32, (TB, B), 1) == seg).astype(jnp.float32)
    ctxg = jnp.dot(onehot, ctx2, preferred_element_type=jnp.float32)
    u = jnp.dot(ev_ref[...], w2a_ref[...], preferred_element_type=jnp.float32)
    h = jax.nn.relu(u + ctxg)
    v = jnp.dot(h, w3_ref[...], preferred_element_type=jnp.float32) + b3_ref[...]
    out_ref[...] = jnp.tanh(v)


def _tc_mlp(ev_emb, seg_col, pooled2, W1, b1, W2a, W2b, b2, W3, b3):
    grid = (T // TB,)
    return pl.pallas_call(
        _tc_mlp_body,
        grid=grid,
        in_specs=[
            pl.BlockSpec((TB, H), lambda i: (i, 0)),
            pl.BlockSpec((TB, 1), lambda i: (i, 0)),
            pl.BlockSpec((B, H), lambda i: (0, 0)),
            pl.BlockSpec((H, H), lambda i: (0, 0)),
            pl.BlockSpec((1, H), lambda i: (0, 0)),
            pl.BlockSpec((H, H), lambda i: (0, 0)),
            pl.BlockSpec((H, H), lambda i: (0, 0)),
            pl.BlockSpec((1, H), lambda i: (0, 0)),
            pl.BlockSpec((H, 1), lambda i: (0, 0)),
            pl.BlockSpec((1, 1), lambda i: (0, 0)),
        ],
        out_specs=pl.BlockSpec((TB, 1), lambda i: (i, 0)),
        out_shape=jax.ShapeDtypeStruct((T, 1), jnp.float32),
    )(ev_emb, seg_col, pooled2, W1, b1, W2a, W2b, b2, W3, b3)


CH = 2048
PAD = 2 * CH

@functools.cache
def _make_sc_scatter():
    mesh = plsc.VectorSubcoreMesh(
        core_axis_name="c", subcore_axis_name="s", num_cores=NC, num_subcores=NS)
    return functools.partial(
        pl.kernel,
        out_type=jax.ShapeDtypeStruct((B, ACT_DIM), jnp.float32),
        mesh=mesh,
        compiler_params=pltpu.CompilerParams(needs_layout_passes=False),
        scratch_types=[
            pltpu.VMEM((LANES,), jnp.int32),
            pltpu.VMEM((CH + LANES,), jnp.int32),
            pltpu.VMEM((CH,), jnp.float32),
            pltpu.VMEM((ACT_DIM,), jnp.float32),
        ],
    )(_sc_scatter_body)


def _sc_scatter_body(vals_hbm, map_hbm, offs_hbm, out_hbm, offs_v, idx_v, vals_v, row_v):
    wid = lax.axis_index("s") * NC + lax.axis_index("c")

    @pl.when(wid < B)
    def _work():
        s = wid
        pltpu.sync_copy(offs_hbm.at[s], offs_v)
        o = offs_v[pl.ds(0, LANES)]
        start = o[0]
        end = o[1]
        lane = lax.broadcasted_iota(jnp.int32, (LANES,), 0)
        abase = (start // 8) * 8
        nch = (end - abase + CH - 1) // CH

        zero16 = jnp.zeros((LANES,), jnp.float32)

        def _zero(i, carry):
            row_v[pl.ds(i * LANES, LANES)] = zero16
            return carry

        lax.fori_loop(0, ACT_DIM // LANES, _zero, 0)

        def _chunk(c, carry):
            cb = abase + c * CH
            pltpu.sync_copy(map_hbm.at[pl.ds(cb, CH + LANES)], idx_v)
            pltpu.sync_copy(vals_hbm.at[pl.ds(cb, CH)], vals_v)

            def _group(g, carry2):
                base = g * LANES
                pos = cb + base + lane
                idx16 = idx_v[pl.ds(base, LANES)]
                val16 = vals_v[pl.ds(base, LANES)]
                valid = (pos >= start) & (pos < end)
                dup = lane < 0
                for sft in range(1, LANES):
                    nidx = idx_v[pl.ds(base + sft, LANES)]
                    dup = dup | (((pos + sft) < end) & (nidx == idx16))
                plsc.store_scatter(row_v, [idx16], val16, mask=valid & (~dup))
                return carry2

            lax.fori_loop(0, CH // LANES, _group, 0)
            return carry

        lax.fori_loop(0, nch, _chunk, 0)
        pltpu.sync_copy(row_v, out_hbm.at[s])


def kernel(pooled, node_embeddings, ev_indexes, segment_ids, action_mapper,
           W1, b1, W2, b2, W3, b3):
    pooled2 = pooled[:, 0, :]
    W2a = W2[:H]
    W2b = W2[H:]
    idx2d = ev_indexes.reshape(T // GATHER_BATCH, GATHER_BATCH)
    seg_col = segment_ids.reshape(T, 1)

    ev_emb = _make_sc_gather()(node_embeddings, idx2d)

    vals = _tc_mlp(ev_emb, seg_col, pooled2, W1, b1.reshape(1, H), W2a, W2b,
                   b2.reshape(1, H), W3, b3.reshape(1, 1))[:, 0]

    offs = jnp.searchsorted(segment_ids,
                            jnp.arange(B + 1, dtype=jnp.int32)).astype(jnp.int32)
    offs2 = jnp.zeros((NW, LANES), jnp.int32)
    offs2 = offs2.at[:B, 0].set(offs[:B]).at[:B, 1].set(offs[1:B + 1])
    vals_pad = jnp.pad(vals, (0, PAD))
    map_pad = jnp.pad(action_mapper, (0, PAD))

    return _make_sc_scatter()(vals_pad, map_pad, offs2)


# speedup vs baseline: 5.7947x; 1.3567x over previous
import functools

import jax
import jax.numpy as jnp
from jax import lax
from jax.experimental import pallas as pl
from jax.experimental.pallas import tpu as pltpu
from jax.experimental.pallas import tpu_sc as plsc

B = 16
H = 128
ACT_DIM = 4096
T = 32768
N_NODES = 65536

NC = 2
NS = 16
NW = NC * NS
LANES = 16


ROWS_PER_W = T // NW
GATHER_BATCH = 128
N_BATCH = ROWS_PER_W // GATHER_BATCH

@functools.cache
def _make_sc_gather():
    mesh = plsc.VectorSubcoreMesh(
        core_axis_name="c", subcore_axis_name="s", num_cores=NC, num_subcores=NS)
    return functools.partial(
        pl.kernel,
        out_type=jax.ShapeDtypeStruct((T, H), jnp.float32),
        mesh=mesh,
        compiler_params=pltpu.CompilerParams(needs_layout_passes=False),
        scratch_types=[
            pltpu.VMEM((N_BATCH, GATHER_BATCH), jnp.int32),
            pltpu.VMEM((GATHER_BATCH, H), jnp.float32),
            pltpu.VMEM((GATHER_BATCH, H), jnp.float32),
            pltpu.SemaphoreType.DMA,
            pltpu.SemaphoreType.DMA,
        ],
    )(_sc_gather_body)


def _sc_gather_body(table_hbm, idx_hbm, out_hbm, idx_v, rows_a, rows_b, sem_a, sem_b):
    wid = lax.axis_index("s") * NC + lax.axis_index("c")
    pltpu.sync_copy(idx_hbm.at[pl.ds(wid * N_BATCH, N_BATCH)], idx_v)
    bufs = (rows_a, rows_b)
    sems = (sem_a, sem_b)
    cps = [None, None]
    cps[0] = pltpu.async_copy(table_hbm.at[idx_v.at[0]], bufs[0], sems[0])
    for j in range(N_BATCH):
        if j + 1 < N_BATCH:
            cps[(j + 1) % 2] = pltpu.async_copy(
                table_hbm.at[idx_v.at[j + 1]], bufs[(j + 1) % 2], sems[(j + 1) % 2])
        cps[j % 2].wait()
        pltpu.sync_copy(
            bufs[j % 2],
            out_hbm.at[pl.ds(wid * ROWS_PER_W + j * GATHER_BATCH, GATHER_BATCH)])


TB = 2048


NB = T // TB


def _tc_mlp_body(ev_ref, seg_ref, pooled_ref, w1_ref, b1_ref, w2a_ref, w2b_ref,
                 b2_ref, w3_ref, b3_ref, out_ref, offs_ref, acc_ref):
    i = pl.program_id(0)
    ctx = jax.nn.relu(
        jnp.dot(pooled_ref[...], w1_ref[...], preferred_element_type=jnp.float32)
        + b1_ref[...])
    ctx2 = jnp.dot(ctx, w2b_ref[...], preferred_element_type=jnp.float32) + b2_ref[...]
    seg = seg_ref[...].reshape(1, TB)
    onehot_t = (lax.broadcasted_iota(jnp.int32, (B, TB), 0) == seg
                ).astype(jnp.float32)
    ctxg = lax.dot_general(onehot_t, ctx2, (((0,), (0,)), ((), ())),
                           preferred_element_type=jnp.float32)
    u = jnp.dot(ev_ref[...], w2a_ref[...], preferred_element_type=jnp.float32)
    h = jax.nn.relu(u + ctxg)
    vt = lax.dot_general(w3_ref[...], h, (((0,), (1,)), ((), ())),
                         preferred_element_type=jnp.float32)
    out_ref[...] = jnp.tanh(vt + b3_ref[...])

    counts = jnp.sum(onehot_t, axis=1, keepdims=True)

    @pl.when(i == 0)
    def _init():
        acc_ref[...] = counts

    @pl.when(i > 0)
    def _acc():
        acc_ref[...] = acc_ref[...] + counts

    @pl.when(i == NB - 1)
    def _fin():
        acc = acc_ref[...]
        tri = (lax.broadcasted_iota(jnp.int32, (B, B), 0)
               > lax.broadcasted_iota(jnp.int32, (B, B), 1)).astype(jnp.float32)
        starts = jnp.dot(tri, acc, preferred_element_type=jnp.float32,
                         precision=lax.Precision.HIGHEST)
        ends = starts + acc
        z = jnp.zeros((NW - B, 1), jnp.float32)
        starts_p = jnp.concatenate([starts, z], axis=0)
        ends_p = jnp.concatenate([ends, z], axis=0)
        col = lax.broadcasted_iota(jnp.int32, (NW, LANES), 1)
        out2 = jnp.where(col == 0, starts_p, jnp.where(col == 1, ends_p, 0.0))
        offs_ref[...] = out2.astype(jnp.int32)


def _tc_mlp(ev_emb, seg3, pooled2, W1, b1, W2a, W2b, b2, W3, b3):
    return pl.pallas_call(
        _tc_mlp_body,
        grid=(NB,),
        in_specs=[
            pl.BlockSpec((TB, H), lambda i: (i, 0)),
            pl.BlockSpec((1, 1, TB), lambda i: (i, 0, 0)),
            pl.BlockSpec((B, H), lambda i: (0, 0)),
            pl.BlockSpec((H, H), lambda i: (0, 0)),
            pl.BlockSpec((1, H), lambda i: (0, 0)),
            pl.BlockSpec((H, H), lambda i: (0, 0)),
            pl.BlockSpec((H, H), lambda i: (0, 0)),
            pl.BlockSpec((1, H), lambda i: (0, 0)),
            pl.BlockSpec((H, 1), lambda i: (0, 0)),
            pl.BlockSpec((1, 1), lambda i: (0, 0)),
        ],
        out_specs=[
            pl.BlockSpec((1, TB), lambda i: (0, i)),
            pl.BlockSpec((NW, LANES), lambda i: (0, 0)),
        ],
        out_shape=[
            jax.ShapeDtypeStruct((1, T), jnp.float32),
            jax.ShapeDtypeStruct((NW, LANES), jnp.int32),
        ],
        scratch_shapes=[pltpu.VMEM((B, 1), jnp.float32)],
    )(ev_emb, seg3, pooled2, W1, b1, W2a, W2b, b2, W3, b3)


CH = 2048
PAD = 2 * CH

@functools.cache
def _make_sc_scatter():
    mesh = plsc.VectorSubcoreMesh(
        core_axis_name="c", subcore_axis_name="s", num_cores=NC, num_subcores=NS)
    return functools.partial(
        pl.kernel,
        out_type=jax.ShapeDtypeStruct((B, ACT_DIM), jnp.float32),
        mesh=mesh,
        compiler_params=pltpu.CompilerParams(needs_layout_passes=False),
        scratch_types=[
            pltpu.VMEM((LANES,), jnp.int32),
            pltpu.VMEM((CH + LANES,), jnp.int32),
            pltpu.VMEM((CH,), jnp.float32),
            pltpu.VMEM((ACT_DIM,), jnp.float32),
        ],
    )(_sc_scatter_body)


def _sc_scatter_body(vals_hbm, map_hbm, offs_hbm, out_hbm, offs_v, idx_v, vals_v, row_v):
    wid = lax.axis_index("s") * NC + lax.axis_index("c")

    @pl.when(wid < B)
    def _work():
        s = wid
        pltpu.sync_copy(offs_hbm.at[s], offs_v)
        o = offs_v[pl.ds(0, LANES)]
        start = o[0]
        end = o[1]
        lane = lax.broadcasted_iota(jnp.int32, (LANES,), 0)
        abase = (start // 8) * 8
        nch = (end - abase + CH - 1) // CH

        zero16 = jnp.zeros((LANES,), jnp.float32)

        def _zero(i, carry):
            row_v[pl.ds(i * LANES, LANES)] = zero16
            return carry

        lax.fori_loop(0, ACT_DIM // LANES, _zero, 0)

        def _chunk(c, carry):
            cb = abase + c * CH
            pltpu.sync_copy(map_hbm.at[pl.ds(cb, CH + LANES)], idx_v)
            pltpu.sync_copy(vals_hbm.at[pl.ds(cb, CH)], vals_v)

            def _group(g, carry2):
                base = g * LANES
                pos = cb + base + lane
                idx16 = idx_v[pl.ds(base, LANES)]
                val16 = vals_v[pl.ds(base, LANES)]
                valid = (pos >= start) & (pos < end)
                dup = lane < 0
                for sft in range(1, LANES):
                    nidx = idx_v[pl.ds(base + sft, LANES)]
                    dup = dup | (((pos + sft) < end) & (nidx == idx16))
                plsc.store_scatter(row_v, [idx16], val16, mask=valid & (~dup))
                return carry2

            lax.fori_loop(0, CH // LANES, _group, 0)
            return carry

        lax.fori_loop(0, nch, _chunk, 0)
        pltpu.sync_copy(row_v, out_hbm.at[s])


def kernel(pooled, node_embeddings, ev_indexes, segment_ids, action_mapper,
           W1, b1, W2, b2, W3, b3):
    pooled2 = pooled[:, 0, :]
    W2a = W2[:H]
    W2b = W2[H:]
    idx2d = ev_indexes.reshape(T // GATHER_BATCH, GATHER_BATCH)
    seg3 = segment_ids.reshape(NB, 1, TB)

    ev_emb = _make_sc_gather()(node_embeddings, idx2d)

    vals_t, offs2 = _tc_mlp(ev_emb, seg3, pooled2, W1, b1.reshape(1, H), W2a,
                            W2b, b2.reshape(1, H), W3, b3.reshape(1, 1))

    vals_pad = jnp.pad(vals_t.reshape(T), (0, PAD))
    map_pad = jnp.pad(action_mapper, (0, PAD))

    return _make_sc_scatter()(vals_pad, map_pad, offs2)


# speedup vs baseline: 5.8982x; 1.0179x over previous
import functools

import jax
import jax.numpy as jnp
from jax import lax
from jax.experimental import pallas as pl
from jax.experimental.pallas import tpu as pltpu
from jax.experimental.pallas import tpu_sc as plsc

B = 16
H = 128
ACT_DIM = 4096
T = 32768
N_NODES = 65536

NC = 2
NS = 16
NW = NC * NS
LANES = 16


ROWS_PER_W = T // NW
GATHER_BATCH = 128
N_BATCH = ROWS_PER_W // GATHER_BATCH

@functools.cache
def _make_sc_gather():
    mesh = plsc.VectorSubcoreMesh(
        core_axis_name="c", subcore_axis_name="s", num_cores=NC, num_subcores=NS)
    return functools.partial(
        pl.kernel,
        out_type=jax.ShapeDtypeStruct((T, H), jnp.float32),
        mesh=mesh,
        compiler_params=pltpu.CompilerParams(needs_layout_passes=False),
        scratch_types=[
            pltpu.VMEM((N_BATCH, GATHER_BATCH), jnp.int32),
            pltpu.VMEM((GATHER_BATCH, H), jnp.float32),
            pltpu.VMEM((GATHER_BATCH, H), jnp.float32),
            pltpu.SemaphoreType.DMA,
            pltpu.SemaphoreType.DMA,
        ],
    )(_sc_gather_body)


def _sc_gather_body(table_hbm, idx_hbm, out_hbm, idx_v, rows_a, rows_b, sem_a, sem_b):
    wid = lax.axis_index("s") * NC + lax.axis_index("c")
    pltpu.sync_copy(idx_hbm.at[pl.ds(wid * N_BATCH, N_BATCH)], idx_v)
    bufs = (rows_a, rows_b)
    sems = (sem_a, sem_b)
    cps = [None, None]
    cps[0] = pltpu.async_copy(table_hbm.at[idx_v.at[0]], bufs[0], sems[0])
    for j in range(N_BATCH):
        if j + 1 < N_BATCH:
            cps[(j + 1) % 2] = pltpu.async_copy(
                table_hbm.at[idx_v.at[j + 1]], bufs[(j + 1) % 2], sems[(j + 1) % 2])
        cps[j % 2].wait()
        pltpu.sync_copy(
            bufs[j % 2],
            out_hbm.at[pl.ds(wid * ROWS_PER_W + j * GATHER_BATCH, GATHER_BATCH)])


TB = 2048


NB = T // TB


def _tc_mlp_body(ev_ref, seg_ref, pooled_ref, w1_ref, b1_ref, w2a_ref, w2b_ref,
                 b2_ref, w3_ref, b3_ref, out_ref, offs_ref, acc_ref):
    i = pl.program_id(0)
    ctx = jax.nn.relu(
        jnp.dot(pooled_ref[...], w1_ref[...], preferred_element_type=jnp.float32)
        + b1_ref[...])
    ctx2 = jnp.dot(ctx, w2b_ref[...], preferred_element_type=jnp.float32) + b2_ref[...]
    seg = seg_ref[...].reshape(1, TB)
    onehot_t = (lax.broadcasted_iota(jnp.int32, (B, TB), 0) == seg
                ).astype(jnp.float32)
    ctxg = lax.dot_general(onehot_t, ctx2, (((0,), (0,)), ((), ())),
                           preferred_element_type=jnp.float32)
    u = jnp.dot(ev_ref[...], w2a_ref[...], preferred_element_type=jnp.float32)
    h = jax.nn.relu(u + ctxg)
    vt = lax.dot_general(w3_ref[...], h, (((0,), (1,)), ((), ())),
                         preferred_element_type=jnp.float32)
    out_ref[...] = jnp.tanh(vt + b3_ref[...])

    counts = jnp.sum(onehot_t, axis=1, keepdims=True)

    @pl.when(i == 0)
    def _init():
        acc_ref[...] = counts

    @pl.when(i > 0)
    def _acc():
        acc_ref[...] = acc_ref[...] + counts

    @pl.when(i == NB - 1)
    def _fin():
        acc = acc_ref[...]
        tri = (lax.broadcasted_iota(jnp.int32, (B, B), 0)
               > lax.broadcasted_iota(jnp.int32, (B, B), 1)).astype(jnp.float32)
        starts = jnp.dot(tri, acc, preferred_element_type=jnp.float32,
                         precision=lax.Precision.HIGHEST)
        ends = starts + acc
        z = jnp.zeros((NW - B, 1), jnp.float32)
        starts_p = jnp.concatenate([starts, z], axis=0)
        ends_p = jnp.concatenate([ends, z], axis=0)
        col = lax.broadcasted_iota(jnp.int32, (NW, LANES), 1)
        out2 = jnp.where(col == 0, starts_p, jnp.where(col == 1, ends_p, 0.0))
        offs_ref[...] = out2.astype(jnp.int32)


def _tc_mlp(ev_emb, seg3, pooled2, W1, b1, W2a, W2b, b2, W3, b3):
    return pl.pallas_call(
        _tc_mlp_body,
        grid=(NB,),
        in_specs=[
            pl.BlockSpec((TB, H), lambda i: (i, 0)),
            pl.BlockSpec((1, 1, TB), lambda i: (i, 0, 0)),
            pl.BlockSpec((B, H), lambda i: (0, 0)),
            pl.BlockSpec((H, H), lambda i: (0, 0)),
            pl.BlockSpec((1, H), lambda i: (0, 0)),
            pl.BlockSpec((H, H), lambda i: (0, 0)),
            pl.BlockSpec((H, H), lambda i: (0, 0)),
            pl.BlockSpec((1, H), lambda i: (0, 0)),
            pl.BlockSpec((H, 1), lambda i: (0, 0)),
            pl.BlockSpec((1, 1), lambda i: (0, 0)),
        ],
        out_specs=[
            pl.BlockSpec((1, TB), lambda i: (0, i)),
            pl.BlockSpec((NW, LANES), lambda i: (0, 0)),
        ],
        out_shape=[
            jax.ShapeDtypeStruct((1, T), jnp.float32),
            jax.ShapeDtypeStruct((NW, LANES), jnp.int32),
        ],
        scratch_shapes=[pltpu.VMEM((B, 1), jnp.float32)],
    )(ev_emb, seg3, pooled2, W1, b1, W2a, W2b, b2, W3, b3)


CH = 2048
PAD = 2 * CH

@functools.cache
def _make_sc_scatter():
    mesh = plsc.VectorSubcoreMesh(
        core_axis_name="c", subcore_axis_name="s", num_cores=NC, num_subcores=NS)
    return functools.partial(
        pl.kernel,
        out_type=jax.ShapeDtypeStruct((B, ACT_DIM), jnp.float32),
        mesh=mesh,
        compiler_params=pltpu.CompilerParams(needs_layout_passes=False),
        scratch_types=[
            pltpu.VMEM((LANES,), jnp.int32),
            pltpu.VMEM((CH + LANES,), jnp.int32),
            pltpu.VMEM((CH,), jnp.float32),
            pltpu.VMEM((ACT_DIM,), jnp.float32),
        ],
    )(_sc_scatter_body)


def _sc_scatter_body(vals_hbm, map_hbm, offs_hbm, out_hbm, offs_v, idx_v, vals_v, row_v):
    wid = lax.axis_index("s") * NC + lax.axis_index("c")

    @pl.when(wid < B)
    def _work():
        s = wid
        pltpu.sync_copy(offs_hbm.at[s], offs_v)
        o = offs_v[pl.ds(0, LANES)]
        start = o[0]
        end = o[1]
        lane = lax.broadcasted_iota(jnp.int32, (LANES,), 0)
        abase = (start // 8) * 8
        nch = (end - abase + CH - 1) // CH

        zero16 = jnp.zeros((LANES,), jnp.float32)

        def _zero(i, carry):
            row_v[pl.ds(i * LANES, LANES)] = zero16
            return carry

        lax.fori_loop(0, ACT_DIM // LANES, _zero, 0)

        def _chunk(c, carry):
            cb = abase + c * CH
            pltpu.sync_copy(map_hbm.at[pl.ds(cb, CH + LANES)], idx_v)
            pltpu.sync_copy(vals_hbm.at[pl.ds(cb, CH)], vals_v)

            def _group(g, carry2):
                base = g * LANES
                pos = cb + base + lane
                idx16 = idx_v[pl.ds(base, LANES)]
                val16 = vals_v[pl.ds(base, LANES)]
                valid = (pos >= start) & (pos < end)
                _, lastmask = plsc.scan_count(idx16, mask=valid)
                plsc.store_scatter(row_v, [idx16], val16, mask=valid & lastmask)
                return carry2

            lax.fori_loop(0, CH // LANES, _group, 0)
            return carry

        lax.fori_loop(0, nch, _chunk, 0)
        pltpu.sync_copy(row_v, out_hbm.at[s])


def kernel(pooled, node_embeddings, ev_indexes, segment_ids, action_mapper,
           W1, b1, W2, b2, W3, b3):
    pooled2 = pooled[:, 0, :]
    W2a = W2[:H]
    W2b = W2[H:]
    idx2d = ev_indexes.reshape(T // GATHER_BATCH, GATHER_BATCH)
    seg3 = segment_ids.reshape(NB, 1, TB)

    ev_emb = _make_sc_gather()(node_embeddings, idx2d)

    vals_t, offs2 = _tc_mlp(ev_emb, seg3, pooled2, W1, b1.reshape(1, H), W2a,
                            W2b, b2.reshape(1, H), W3, b3.reshape(1, 1))

    vals_pad = jnp.pad(vals_t.reshape(T), (0, PAD))
    map_pad = jnp.pad(action_mapper, (0, PAD))

    return _make_sc_scatter()(vals_pad, map_pad, offs2)
